# Initial kernel scaffold; baseline (speedup 1.0000x reference)
import jax
import jax.numpy as jnp
from jax.experimental import pallas as pl


def kernel(x0, x1, x2, x3, x4, x5, x6, x7, x8, x9, edge_index, edge_weight, W, b):
    raise NotImplementedError("write your pallas kernel here")


# speedup vs baseline: 3.8565x; 3.8565x over previous
import functools

import jax
import jax.numpy as jnp
from jax import lax
from jax.experimental import pallas as pl
from jax.experimental.pallas import tpu as pltpu
from jax.experimental.pallas import tpu_sc as plsc

N = 10000
E = 320000
D = 128
NCH = 10

NCORES = 2
NSUB = 16
NW = NCORES * NSUB
L = 16

ES = E // NW
F = 4
FB = D // F
NTASK = NCH * FB
TT = NTASK // NW
CE = 8000
NCHUNK = E // CE
GC = CE // L

_mesh = plsc.VectorSubcoreMesh(core_axis_name="c", subcore_axis_name="s")
_sc_params = pltpu.CompilerParams(needs_layout_passes=False)


def _wid():
    return lax.axis_index("s") * NCORES + lax.axis_index("c")


@functools.partial(
    pl.kernel,
    out_type=jax.ShapeDtypeStruct((NW, N), jnp.float32),
    mesh=_mesh,
    compiler_params=_sc_params,
    scratch_types=[
        pltpu.VMEM((N,), jnp.float32),
        pltpu.VMEM((ES,), jnp.int32),
        pltpu.VMEM((ES,), jnp.float32),
    ],
)
def _deg_kernel(col_hbm, ew_hbm, deg_out, deg_v, col_v, ew_v):
    wid = _wid()
    base = wid * ES
    pltpu.sync_copy(col_hbm.at[pl.ds(base, ES)], col_v)
    pltpu.sync_copy(ew_hbm.at[pl.ds(base, ES)], ew_v)

    def zbody(i, _):
        deg_v[pl.ds(i * L, L)] = jnp.zeros((L,), jnp.float32)
        return 0

    lax.fori_loop(0, N // L, zbody, 0)

    def ebody(i, _):
        sl = pl.ds(i * L, L)
        plsc.addupdate_scatter(deg_v, [col_v[sl]], ew_v[sl])
        return 0

    lax.fori_loop(0, ES // L, ebody, 0)
    pltpu.sync_copy(deg_v, deg_out.at[wid])


def _dis_body(degp_ref, dis_ref, dissq_ref):
    deg = jnp.sum(degp_ref[...], axis=0) + 1.0
    dis = jnp.where(deg > 0, lax.rsqrt(jnp.maximum(deg, 1e-12)), 0.0)
    dis_ref[...] = dis
    dissq_ref[...] = dis * dis


def _dis_call(deg_part):
    return pl.pallas_call(
        _dis_body,
        out_shape=(
            jax.ShapeDtypeStruct((N,), jnp.float32),
            jax.ShapeDtypeStruct((N,), jnp.float32),
        ),
    )(deg_part)


def _mm_body(wt_ref, x_ref, ht_ref):
    ht_ref[0] = lax.dot_general(
        wt_ref[0], x_ref[0],
        dimension_numbers=(((1,), (1,)), ((), ())),
        preferred_element_type=jnp.float32,
    )


def _mm_call(WT, X):
    return pl.pallas_call(
        _mm_body,
        grid=(NCH,),
        in_specs=[
            pl.BlockSpec((1, D, D), lambda i: (i, 0, 0)),
            pl.BlockSpec((1, N, D), lambda i: (i, 0, 0)),
        ],
        out_specs=pl.BlockSpec((1, D, N), lambda i: (i, 0, 0)),
        out_shape=jax.ShapeDtypeStruct((NCH, D, N), jnp.float32),
    )(WT, X)


@functools.partial(
    pl.kernel,
    out_type=jax.ShapeDtypeStruct((E,), jnp.float32),
    mesh=_mesh,
    compiler_params=_sc_params,
    scratch_types=[
        pltpu.VMEM((N,), jnp.float32),
        pltpu.VMEM((ES,), jnp.int32),
        pltpu.VMEM((ES,), jnp.int32),
        pltpu.VMEM((ES,), jnp.float32),
        pltpu.VMEM((ES,), jnp.float32),
    ],
)
def _norm_kernel(row_hbm, col_hbm, ew_hbm, dis_hbm, norm_out,
                 dis_v, row_v, col_v, ew_v, norm_v):
    wid = _wid()
    base = wid * ES
    pltpu.sync_copy(dis_hbm, dis_v)
    pltpu.sync_copy(row_hbm.at[pl.ds(base, ES)], row_v)
    pltpu.sync_copy(col_hbm.at[pl.ds(base, ES)], col_v)
    pltpu.sync_copy(ew_hbm.at[pl.ds(base, ES)], ew_v)

    def body(i, _):
        sl = pl.ds(i * L, L)
        dr = plsc.load_gather(dis_v, [row_v[sl]])
        dc = plsc.load_gather(dis_v, [col_v[sl]])
        norm_v[sl] = dr * ew_v[sl] * dc
        return 0

    lax.fori_loop(0, ES // L, body, 0)
    pltpu.sync_copy(norm_v, norm_out.at[pl.ds(base, ES)])


@functools.partial(
    pl.kernel,
    out_type=jax.ShapeDtypeStruct((NCH, D, N), jnp.float32),
    mesh=_mesh,
    compiler_params=_sc_params,
    scratch_types=(
        [pltpu.VMEM((N,), jnp.float32)]
        + [pltpu.VMEM((N,), jnp.float32)] * F
        + [pltpu.VMEM((N,), jnp.float32)] * F
        + [
            pltpu.VMEM((CE,), jnp.int32),
            pltpu.VMEM((CE,), jnp.int32),
            pltpu.VMEM((CE,), jnp.float32),
        ]
    ),
)
def _msg_kernel(ht_hbm, row_hbm, col_hbm, norm_hbm, dissq_hbm, mt_out,
                dissq_v, ht0, ht1, ht2, ht3, ac0, ac1, ac2, ac3,
                row_c, col_c, norm_c):
    hts = [ht0, ht1, ht2, ht3]
    accs = [ac0, ac1, ac2, ac3]
    wid = _wid()
    pltpu.sync_copy(dissq_hbm, dissq_v)

    def task(ti, _):
        t = wid * TT + ti
        c = t // FB
        j0 = (t % FB) * F
        for jj in range(F):
            pltpu.sync_copy(ht_hbm.at[c, j0 + jj], hts[jj])

        def init(k, _):
            sl = pl.ds(k * L, L)
            dv = dissq_v[sl]
            for jj in range(F):
                accs[jj][sl] = dv * hts[jj][sl]
            return 0

        lax.fori_loop(0, N // L, init, 0)

        def chunk(ch, _):
            cb = ch * CE
            pltpu.sync_copy(row_hbm.at[pl.ds(cb, CE)], row_c)
            pltpu.sync_copy(col_hbm.at[pl.ds(cb, CE)], col_c)
            pltpu.sync_copy(norm_hbm.at[pl.ds(cb, CE)], norm_c)

            def group(g, _):
                sl = pl.ds(g * L, L)
                rv = row_c[sl]
                cv = col_c[sl]
                nv = norm_c[sl]
                for jj in range(F):
                    hv = plsc.load_gather(hts[jj], [rv])
                    plsc.addupdate_scatter(accs[jj], [cv], hv * nv)
                return 0

            lax.fori_loop(0, GC, group, 0)
            return 0

        lax.fori_loop(0, NCHUNK, chunk, 0)
        for jj in range(F):
            pltpu.sync_copy(accs[jj], mt_out.at[c, j0 + jj])
        return 0

    lax.fori_loop(0, TT, task, 0)


def _fin_body(mt_ref, b_ref, out_ref):
    mt = mt_ref[0]
    ii = lax.broadcasted_iota(jnp.int32, (D, D), 0)
    jj = lax.broadcasted_iota(jnp.int32, (D, D), 1)
    eye = jnp.where(ii == jj, 1.0, 0.0).astype(jnp.float32)
    out = lax.dot_general(
        mt, eye,
        dimension_numbers=(((0,), (0,)), ((), ())),
        preferred_element_type=jnp.float32,
    )
    out_ref[0] = out + b_ref[pl.program_id(0)][None, :]


def _fin_call(MT, b):
    return pl.pallas_call(
        _fin_body,
        grid=(NCH,),
        in_specs=[
            pl.BlockSpec((1, D, N), lambda i: (i, 0, 0)),
            pl.BlockSpec((NCH, D), lambda i: (0, 0)),
        ],
        out_specs=pl.BlockSpec((1, N, D), lambda i: (i, 0, 0)),
        out_shape=jax.ShapeDtypeStruct((NCH, N, D), jnp.float32),
    )(MT, b)


def kernel(x0, x1, x2, x3, x4, x5, x6, x7, x8, x9, edge_index, edge_weight, W, b):
    X = jnp.stack([x0, x1, x2, x3, x4, x5, x6, x7, x8, x9])
    row = edge_index[0]
    col = edge_index[1]
    WT = jnp.swapaxes(W, 1, 2)

    deg_part = _deg_kernel(col, edge_weight)
    dis, dissq = _dis_call(deg_part)
    HT = _mm_call(WT, X)
    norm = _norm_kernel(row, col, edge_weight, dis)
    MT = _msg_kernel(HT, row, col, norm, dissq)
    OUT = _fin_call(MT, b)
    return tuple(OUT[i] for i in range(NCH))
